# Initial kernel scaffold; baseline (speedup 1.0000x reference)
import jax
import jax.numpy as jnp
from jax.experimental import pallas as pl


def kernel(x, Wg, bg, W1, b1, W2, b2):
    raise NotImplementedError("write your pallas kernel here")


# speedup vs baseline: 4.0876x; 4.0876x over previous
import functools
import math

import jax
import jax.numpy as jnp
from jax import lax
from jax.experimental import pallas as pl
from jax.experimental.pallas import tpu as pltpu

D_MODEL = 768
D_FF = 3072
N_EXP = 8
SEQ = 2048
BT = 256


def _moe_body(xb_ref, wg_ref, bg_ref, w1_ref, b1_ref, w2_ref, b2_ref,
              out_ref, c_ref):
    e = pl.program_id(0)
    tb = pl.program_id(1)
    rows = pl.ds(tb * BT, BT)

    @pl.when(e == 0)
    def _gating():
        logits = jnp.dot(xb_ref[...], wg_ref[...],
                         preferred_element_type=jnp.float32) + bg_ref[...]
        lane = lax.broadcasted_iota(jnp.int32, (BT, N_EXP), 1)
        m1 = jnp.max(logits, axis=-1, keepdims=True)
        a1 = jnp.min(jnp.where(logits == m1, lane, N_EXP), axis=-1,
                     keepdims=True)
        l2 = jnp.where(lane == a1, -jnp.inf, logits)
        m2 = jnp.max(l2, axis=-1, keepdims=True)
        a2 = jnp.min(jnp.where(l2 == m2, lane, N_EXP), axis=-1,
                     keepdims=True)
        ed = jnp.exp(m2 - m1)
        s1 = 1.0 / (1.0 + ed)
        s2 = ed / (1.0 + ed)
        c = (jnp.where(lane == a1, s1, 0.0)
             + jnp.where(lane == a2, s2, 0.0))
        c_ref[rows, :] = c
        out_ref[rows, :] = xb_ref[...]

    xb = xb_ref[...].astype(jnp.bfloat16)
    w1 = w1_ref[0].astype(jnp.bfloat16)
    h = jnp.dot(xb, w1, preferred_element_type=jnp.float32) + b1_ref[0]
    h = h * 0.5 * (1.0 + lax.erf(h * (1.0 / math.sqrt(2.0))))
    w2 = w2_ref[0].astype(jnp.bfloat16)
    y = (jnp.dot(h.astype(jnp.bfloat16), w2,
                 preferred_element_type=jnp.float32) + b2_ref[0])
    lane = lax.broadcasted_iota(jnp.int32, (BT, N_EXP), 1)
    ce = jnp.sum(jnp.where(lane == e, c_ref[rows, :], 0.0), axis=-1,
                 keepdims=True)
    out_ref[rows, :] += ce * y


@jax.jit
def _moe(x_flat, Wg, bg2, W1, b1, W2, b2):
    n_tb = SEQ // BT
    return pl.pallas_call(
        _moe_body,
        grid=(N_EXP, n_tb),
        in_specs=[
            pl.BlockSpec((BT, D_MODEL), lambda e, t: (t, 0)),
            pl.BlockSpec((D_MODEL, N_EXP), lambda e, t: (0, 0)),
            pl.BlockSpec((1, N_EXP), lambda e, t: (0, 0)),
            pl.BlockSpec((1, D_MODEL, D_FF), lambda e, t: (e, 0, 0)),
            pl.BlockSpec((1, 1, D_FF), lambda e, t: (e, 0, 0)),
            pl.BlockSpec((1, D_FF, D_MODEL), lambda e, t: (e, 0, 0)),
            pl.BlockSpec((1, 1, D_MODEL), lambda e, t: (e, 0, 0)),
        ],
        out_specs=pl.BlockSpec((SEQ, D_MODEL), lambda e, t: (0, 0)),
        out_shape=jax.ShapeDtypeStruct((SEQ, D_MODEL), jnp.float32),
        scratch_shapes=[pltpu.VMEM((SEQ, N_EXP), jnp.float32)],
        compiler_params=pltpu.CompilerParams(
            dimension_semantics=("arbitrary", "arbitrary"),
            vmem_limit_bytes=120 * 1024 * 1024,
        ),
    )(x_flat, Wg, bg2, W1, b1, W2, b2)


def kernel(x, Wg, bg, W1, b1, W2, b2):
    B, S, D = x.shape
    x_flat = x.reshape(S, D)
    out = _moe(x_flat, Wg, bg.reshape(1, N_EXP),
               W1, b1.reshape(N_EXP, 1, D_FF), W2,
               b2.reshape(N_EXP, 1, D_MODEL))
    return out.reshape(B, S, D)
